# Initial kernel scaffold; baseline (speedup 1.0000x reference)
import functools

import jax
import jax.numpy as jnp
from jax import lax
from jax.experimental import pallas as pl
from jax.experimental.pallas import tpu as pltpu
from jax.experimental.pallas import tpu_sc as plsc

_N, _E, _H, _NG = 10000, 320000, 128, 64
_BN_EPS = 1e-05

_NPAD = 10240
_CH = 128
_NCH = 79
_EPW = _CH * _NCH
_EP = 32 * _EPW
_RPS = _NPAD // 16
_ZPW = _NPAD // 32
_ZCH = 80
_NZC = _ZPW // _ZCH

_BN = 1024
_GRID = _NPAD // _BN

_mesh = plsc.VectorSubcoreMesh(core_axis_name="c", subcore_axis_name="s")


@functools.partial(
    pl.kernel,
    mesh=_mesh,
    out_type=jax.ShapeDtypeStruct((_NPAD, _H), jnp.float32),
    scratch_types=[
        pltpu.VMEM((_ZCH,), jnp.int32),
        pltpu.VMEM((_ZCH, _H), jnp.float32),
        pltpu.SemaphoreType.DMA,
    ],
)
def _embed(tab_hbm, z_hbm, x_hbm, idx_v, rows_v, sem):
    wid = lax.axis_index("s") * 2 + lax.axis_index("c")
    base0 = wid * _ZPW

    def body(j, carry):
        base = base0 + j * _ZCH
        pltpu.sync_copy(z_hbm.at[pl.ds(base, _ZCH)], idx_v)
        pltpu.async_copy(tab_hbm.at[idx_v], rows_v, sem).wait()
        pltpu.sync_copy(rows_v, x_hbm.at[pl.ds(base, _ZCH), :])
        return carry

    lax.fori_loop(0, _NZC, body, 0)


@functools.partial(
    pl.kernel,
    mesh=_mesh,
    out_type=jax.ShapeDtypeStruct((2 * _NPAD, _H), jnp.float32),
    scratch_types=[
        pltpu.VMEM((_CH,), jnp.int32),
        pltpu.VMEM((_CH,), jnp.int32),
        pltpu.VMEM((_CH, _H), jnp.float32),
        pltpu.VMEM((_RPS, _H), jnp.float32),
        pltpu.VMEM_SHARED((_NPAD, _H), jnp.float32),
        pltpu.SemaphoreType.DMA,
    ],
)
def _scatter(x_hbm, src_hbm, dst_hbm, zeros_hbm, out_hbm,
             sidx, didx, rows, zbuf, agg, sem):
    c = lax.axis_index("c")
    s = lax.axis_index("s")
    wid = s * 2 + c
    rbase = s * _RPS
    pltpu.sync_copy(zeros_hbm, zbuf)
    pltpu.sync_copy(zbuf, agg.at[pl.ds(rbase, _RPS), :])
    plsc.subcore_barrier()

    ebase = wid * _EPW

    def body(j, carry):
        base = ebase + j * _CH
        pltpu.sync_copy(src_hbm.at[pl.ds(base, _CH)], sidx)
        pltpu.sync_copy(dst_hbm.at[pl.ds(base, _CH)], didx)
        pltpu.async_copy(x_hbm.at[sidx], rows, sem).wait()
        pltpu.sync_copy(rows, agg.at[didx], add=True)
        return carry

    lax.fori_loop(0, _NCH, body, 0)
    plsc.subcore_barrier()
    obase = c * _NPAD + rbase
    pltpu.sync_copy(agg.at[pl.ds(rbase, _RPS), :],
                    out_hbm.at[pl.ds(obase, _RPS), :])


def _mlp_body(x_ref, agg_ref, w1_ref, b1_ref, w2_ref, b2_ref, sc_ref,
              be_ref, o_ref):
    h = x_ref[...] + agg_ref[0] + agg_ref[1]
    h = jnp.dot(h, w1_ref[...], preferred_element_type=jnp.float32) + b1_ref[...]
    h = jnp.maximum(h, 0.0)
    h = jnp.dot(h, w2_ref[...], preferred_element_type=jnp.float32) + b2_ref[...]
    h = jnp.maximum(h, 0.0)
    o_ref[...] = h * sc_ref[...] + be_ref[...]


def _mlp(x, agg, w1, b1, w2, b2, scl, be):
    return pl.pallas_call(
        _mlp_body,
        grid=(_GRID,),
        in_specs=[
            pl.BlockSpec((_BN, _H), lambda i: (i, 0)),
            pl.BlockSpec((2, _BN, _H), lambda i: (0, i, 0)),
            pl.BlockSpec((_H, _H), lambda i: (0, 0)),
            pl.BlockSpec((1, _H), lambda i: (0, 0)),
            pl.BlockSpec((_H, _H), lambda i: (0, 0)),
            pl.BlockSpec((1, _H), lambda i: (0, 0)),
            pl.BlockSpec((1, _H), lambda i: (0, 0)),
            pl.BlockSpec((1, _H), lambda i: (0, 0)),
        ],
        out_specs=pl.BlockSpec((_BN, _H), lambda i: (i, 0)),
        out_shape=jax.ShapeDtypeStruct((_NPAD, _H), jnp.float32),
    )(x, agg, w1, b1, w2, b2, scl, be)


def _pool_body(x1_ref, x2_ref, x3_ref, b_ref, w1_ref, b1_ref, w2_ref,
               b2_ref, o_ref, sums, cnt):
    i = pl.program_id(0)

    @pl.when(i == 0)
    def _():
        sums[...] = jnp.zeros((_NG, 3 * _H), jnp.float32)
        cnt[...] = jnp.zeros((_NG, _H), jnp.float32)

    seg = b_ref[0, 0, :]
    oh = (lax.broadcasted_iota(jnp.int32, (_NG, _BN), 0)
          == seg[None, :]).astype(jnp.float32)
    sums[:, 0:_H] += jnp.dot(oh, x1_ref[...], preferred_element_type=jnp.float32)
    sums[:, _H:2 * _H] += jnp.dot(oh, x2_ref[...], preferred_element_type=jnp.float32)
    sums[:, 2 * _H:3 * _H] += jnp.dot(oh, x3_ref[...], preferred_element_type=jnp.float32)
    cnt[...] += jnp.broadcast_to(jnp.sum(oh, axis=1, keepdims=True), (_NG, _H))

    @pl.when(i == _GRID - 1)
    def _():
        c = jnp.maximum(cnt[...], 1.0)
        h = (jnp.dot(sums[:, 0:_H] / c, w1_ref[0:_H, :],
                     preferred_element_type=jnp.float32)
             + jnp.dot(sums[:, _H:2 * _H] / c, w1_ref[_H:2 * _H, :],
                       preferred_element_type=jnp.float32)
             + jnp.dot(sums[:, 2 * _H:3 * _H] / c, w1_ref[2 * _H:3 * _H, :],
                       preferred_element_type=jnp.float32)
             + b1_ref[...])
        h = jnp.maximum(h, 0.0)
        o_ref[...] = jnp.dot(h, w2_ref[...],
                             preferred_element_type=jnp.float32) + b2_ref[...]


def _pool(x1, x2, x3, bp, w1, b1, w2p, b2p):
    return pl.pallas_call(
        _pool_body,
        grid=(_GRID,),
        in_specs=[
            pl.BlockSpec((_BN, _H), lambda i: (i, 0)),
            pl.BlockSpec((_BN, _H), lambda i: (i, 0)),
            pl.BlockSpec((_BN, _H), lambda i: (i, 0)),
            pl.BlockSpec((1, 1, _BN), lambda i: (i, 0, 0)),
            pl.BlockSpec((3 * _H, _H), lambda i: (0, 0)),
            pl.BlockSpec((1, _H), lambda i: (0, 0)),
            pl.BlockSpec((_H, _H), lambda i: (0, 0)),
            pl.BlockSpec((1, _H), lambda i: (0, 0)),
        ],
        out_specs=pl.BlockSpec((_NG, _H), lambda i: (0, 0)),
        out_shape=jax.ShapeDtypeStruct((_NG, _H), jnp.float32),
        scratch_shapes=[
            pltpu.VMEM((_NG, 3 * _H), jnp.float32),
            pltpu.VMEM((_NG, _H), jnp.float32),
        ],
    )(x1, x2, x3, bp, w1, b1, w2p, b2p)


def kernel(z, edge_index, batch, z_table, W1_0, b1_0, W2_0, b2_0, g_0, be_0,
           W1_1, b1_1, W2_1, b2_1, g_1, be_1, W1_2, b1_2, W2_2, b2_2, g_2,
           be_2, lin1_W, lin1_b, lin2_W, lin2_b):
    f32 = jnp.float32
    z = z.astype(jnp.int32)
    ei = edge_index.astype(jnp.int32)
    batch = batch.astype(jnp.int32)

    src = jnp.concatenate([ei[0], jnp.zeros((_EP - _E,), jnp.int32)])
    dst = jnp.concatenate([ei[1], jnp.full((_EP - _E,), _NPAD - 1, jnp.int32)])
    zp = jnp.concatenate([z, jnp.zeros((_NPAD - _N,), jnp.int32)])
    zeros_stage = jnp.zeros((_RPS, _H), f32)

    x = _embed(z_table, zp)

    layers = [
        (W1_0, b1_0, W2_0, b2_0, g_0, be_0),
        (W1_1, b1_1, W2_1, b2_1, g_1, be_1),
        (W1_2, b1_2, W2_2, b2_2, g_2, be_2),
    ]
    xs = []
    for (w1, b1, w2, b2, g, be) in layers:
        aggf = _scatter(x, src, dst, zeros_stage)
        agg = aggf.reshape(2, _NPAD, _H)
        scl = (g / jnp.sqrt(1.0 + _BN_EPS)).reshape(1, _H)
        x = _mlp(x, agg, w1, b1.reshape(1, _H), w2, b2.reshape(1, _H),
                 scl, be.reshape(1, _H))
        xs.append(x)

    bp = jnp.concatenate(
        [batch, jnp.full((_NPAD - _N,), _NG, jnp.int32)]).reshape(_GRID, 1, _BN)
    w2p = jnp.pad(lin2_W, ((0, 0), (0, _H - 1)))
    b2p = jnp.pad(lin2_b, (0, _H - 1)).reshape(1, _H)
    out = _pool(xs[0], xs[1], xs[2], bp, lin1_W, lin1_b.reshape(1, _H),
                w2p, b2p)
    return out[:, :1]


# speedup vs baseline: 3.4697x; 3.4697x over previous
import functools

import jax
import jax.numpy as jnp
from jax import lax
from jax.experimental import pallas as pl
from jax.experimental.pallas import tpu as pltpu
from jax.experimental.pallas import tpu_sc as plsc

_N, _E, _H, _NG = 10000, 320000, 128, 64
_BN_EPS = 1e-05

_NPAD = 10240
_CH = 128
_NCH = 79
_EPW = _CH * _NCH
_EP = 32 * _EPW
_RPS = _NPAD // 16
_ZB = 64
_NZB = _RPS // _ZB
_ZPW = _NPAD // 32
_ZCH = 80
_NZC = _ZPW // _ZCH

_BN = 1024
_GRID = _NPAD // _BN

_mesh = plsc.VectorSubcoreMesh(core_axis_name="c", subcore_axis_name="s")


@functools.partial(
    pl.kernel,
    mesh=_mesh,
    out_type=jax.ShapeDtypeStruct((_NPAD, _H), jnp.float32),
    scratch_types=[
        pltpu.VMEM((_ZCH,), jnp.int32),
        pltpu.VMEM((_ZCH, _H), jnp.float32),
        pltpu.SemaphoreType.DMA,
    ],
)
def _embed(tab_hbm, z_hbm, x_hbm, idx_v, rows_v, sem):
    wid = lax.axis_index("s") * 2 + lax.axis_index("c")
    base0 = wid * _ZPW

    def body(j, carry):
        base = base0 + j * _ZCH
        pltpu.sync_copy(z_hbm.at[pl.ds(base, _ZCH)], idx_v)
        pltpu.async_copy(tab_hbm.at[idx_v], rows_v, sem).wait()
        pltpu.sync_copy(rows_v, x_hbm.at[pl.ds(base, _ZCH), :])
        return carry

    lax.fori_loop(0, _NZC, body, 0)


@functools.partial(
    pl.kernel,
    mesh=_mesh,
    out_type=jax.ShapeDtypeStruct((2 * _NPAD, _H), jnp.float32),
    scratch_types=[
        pltpu.VMEM((_CH,), jnp.int32),
        pltpu.VMEM((_CH,), jnp.int32),
        pltpu.VMEM((_CH, _H), jnp.float32),
        pltpu.VMEM((_ZB, _H), jnp.float32),
        pltpu.VMEM_SHARED((_NPAD, _H), jnp.float32),
        pltpu.SemaphoreType.DMA,
    ],
)
def _scatter(x_hbm, src_hbm, dst_hbm, zeros_hbm, out_hbm,
             sidx, didx, rows, zbuf, agg, sem):
    c = lax.axis_index("c")
    s = lax.axis_index("s")
    wid = s * 2 + c
    rbase = s * _RPS
    pltpu.sync_copy(zeros_hbm, zbuf)

    def zbody(j, carry):
        pltpu.sync_copy(zbuf, agg.at[pl.ds(rbase + j * _ZB, _ZB), :])
        return carry

    lax.fori_loop(0, _NZB, zbody, 0)
    plsc.subcore_barrier()

    ebase = wid * _EPW

    def body(j, carry):
        base = ebase + j * _CH
        pltpu.sync_copy(src_hbm.at[pl.ds(base, _CH)], sidx)
        pltpu.sync_copy(dst_hbm.at[pl.ds(base, _CH)], didx)
        pltpu.async_copy(x_hbm.at[sidx], rows, sem).wait()
        pltpu.sync_copy(rows, agg.at[didx], add=True)
        return carry

    lax.fori_loop(0, _NCH, body, 0)
    plsc.subcore_barrier()
    obase = c * _NPAD + rbase
    pltpu.sync_copy(agg.at[pl.ds(rbase, _RPS), :],
                    out_hbm.at[pl.ds(obase, _RPS), :])


def _mlp_body(x_ref, agg_ref, w1_ref, b1_ref, w2_ref, b2_ref, sc_ref,
              be_ref, o_ref):
    h = x_ref[...] + agg_ref[0] + agg_ref[1]
    h = jnp.dot(h, w1_ref[...], preferred_element_type=jnp.float32) + b1_ref[...]
    h = jnp.maximum(h, 0.0)
    h = jnp.dot(h, w2_ref[...], preferred_element_type=jnp.float32) + b2_ref[...]
    h = jnp.maximum(h, 0.0)
    o_ref[...] = h * sc_ref[...] + be_ref[...]


def _mlp(x, agg, w1, b1, w2, b2, scl, be):
    return pl.pallas_call(
        _mlp_body,
        grid=(_GRID,),
        in_specs=[
            pl.BlockSpec((_BN, _H), lambda i: (i, 0)),
            pl.BlockSpec((2, _BN, _H), lambda i: (0, i, 0)),
            pl.BlockSpec((_H, _H), lambda i: (0, 0)),
            pl.BlockSpec((1, _H), lambda i: (0, 0)),
            pl.BlockSpec((_H, _H), lambda i: (0, 0)),
            pl.BlockSpec((1, _H), lambda i: (0, 0)),
            pl.BlockSpec((1, _H), lambda i: (0, 0)),
            pl.BlockSpec((1, _H), lambda i: (0, 0)),
        ],
        out_specs=pl.BlockSpec((_BN, _H), lambda i: (i, 0)),
        out_shape=jax.ShapeDtypeStruct((_NPAD, _H), jnp.float32),
    )(x, agg, w1, b1, w2, b2, scl, be)


def _pool_body(x1_ref, x2_ref, x3_ref, b_ref, w1_ref, b1_ref, w2_ref,
               b2_ref, o_ref, sums, cnt):
    i = pl.program_id(0)

    @pl.when(i == 0)
    def _():
        sums[...] = jnp.zeros((_NG, 3 * _H), jnp.float32)
        cnt[...] = jnp.zeros((_NG, _H), jnp.float32)

    seg = b_ref[0, 0, :]
    oh = (lax.broadcasted_iota(jnp.int32, (_NG, _BN), 0)
          == seg[None, :]).astype(jnp.float32)
    sums[:, 0:_H] += jnp.dot(oh, x1_ref[...], preferred_element_type=jnp.float32)
    sums[:, _H:2 * _H] += jnp.dot(oh, x2_ref[...], preferred_element_type=jnp.float32)
    sums[:, 2 * _H:3 * _H] += jnp.dot(oh, x3_ref[...], preferred_element_type=jnp.float32)
    cnt[...] += jnp.broadcast_to(jnp.sum(oh, axis=1, keepdims=True), (_NG, _H))

    @pl.when(i == _GRID - 1)
    def _():
        c = jnp.maximum(cnt[...], 1.0)
        h = (jnp.dot(sums[:, 0:_H] / c, w1_ref[0:_H, :],
                     preferred_element_type=jnp.float32)
             + jnp.dot(sums[:, _H:2 * _H] / c, w1_ref[_H:2 * _H, :],
                       preferred_element_type=jnp.float32)
             + jnp.dot(sums[:, 2 * _H:3 * _H] / c, w1_ref[2 * _H:3 * _H, :],
                       preferred_element_type=jnp.float32)
             + b1_ref[...])
        h = jnp.maximum(h, 0.0)
        o_ref[...] = jnp.dot(h, w2_ref[...],
                             preferred_element_type=jnp.float32) + b2_ref[...]


def _pool(x1, x2, x3, bp, w1, b1, w2p, b2p):
    return pl.pallas_call(
        _pool_body,
        grid=(_GRID,),
        in_specs=[
            pl.BlockSpec((_BN, _H), lambda i: (i, 0)),
            pl.BlockSpec((_BN, _H), lambda i: (i, 0)),
            pl.BlockSpec((_BN, _H), lambda i: (i, 0)),
            pl.BlockSpec((1, 1, _BN), lambda i: (i, 0, 0)),
            pl.BlockSpec((3 * _H, _H), lambda i: (0, 0)),
            pl.BlockSpec((1, _H), lambda i: (0, 0)),
            pl.BlockSpec((_H, _H), lambda i: (0, 0)),
            pl.BlockSpec((1, _H), lambda i: (0, 0)),
        ],
        out_specs=pl.BlockSpec((_NG, _H), lambda i: (0, 0)),
        out_shape=jax.ShapeDtypeStruct((_NG, _H), jnp.float32),
        scratch_shapes=[
            pltpu.VMEM((_NG, 3 * _H), jnp.float32),
            pltpu.VMEM((_NG, _H), jnp.float32),
        ],
    )(x1, x2, x3, bp, w1, b1, w2p, b2p)


def kernel(z, edge_index, batch, z_table, W1_0, b1_0, W2_0, b2_0, g_0, be_0,
           W1_1, b1_1, W2_1, b2_1, g_1, be_1, W1_2, b1_2, W2_2, b2_2, g_2,
           be_2, lin1_W, lin1_b, lin2_W, lin2_b):
    f32 = jnp.float32
    z = z.astype(jnp.int32)
    ei = edge_index.astype(jnp.int32)
    batch = batch.astype(jnp.int32)

    src = jnp.concatenate([ei[0], jnp.zeros((_EP - _E,), jnp.int32)])
    dst = jnp.concatenate([ei[1], jnp.full((_EP - _E,), _NPAD - 1, jnp.int32)])
    zp = jnp.concatenate([z, jnp.zeros((_NPAD - _N,), jnp.int32)])
    zeros_stage = jnp.zeros((_ZB, _H), f32)

    x = _embed(z_table, zp)

    layers = [
        (W1_0, b1_0, W2_0, b2_0, g_0, be_0),
        (W1_1, b1_1, W2_1, b2_1, g_1, be_1),
        (W1_2, b1_2, W2_2, b2_2, g_2, be_2),
    ]
    xs = []
    for (w1, b1, w2, b2, g, be) in layers:
        aggf = _scatter(x, src, dst, zeros_stage)
        agg = aggf.reshape(2, _NPAD, _H)
        scl = (g / jnp.sqrt(1.0 + _BN_EPS)).reshape(1, _H)
        x = _mlp(x, agg, w1, b1.reshape(1, _H), w2, b2.reshape(1, _H),
                 scl, be.reshape(1, _H))
        xs.append(x)

    bp = jnp.concatenate(
        [batch, jnp.full((_NPAD - _N,), _NG, jnp.int32)]).reshape(_GRID, 1, _BN)
    w2p = jnp.pad(lin2_W, ((0, 0), (0, _H - 1)))
    b2p = jnp.pad(lin2_b, (0, _H - 1)).reshape(1, _H)
    out = _pool(xs[0], xs[1], xs[2], bp, lin1_W, lin1_b.reshape(1, _H),
                w2p, b2p)
    return out[:, :1]


# speedup vs baseline: 4.4060x; 1.2699x over previous
import functools

import jax
import jax.numpy as jnp
from jax import lax
from jax.experimental import pallas as pl
from jax.experimental.pallas import tpu as pltpu
from jax.experimental.pallas import tpu_sc as plsc

_N, _E, _H, _NG = 10000, 320000, 128, 64
_BN_EPS = 1e-05

_NPAD = 10240
_CH = 128
_NCH = 79
_EPW = _CH * _NCH
_EP = 32 * _EPW
_RPS = _NPAD // 16
_ZB = 64
_NZB = _RPS // _ZB
_ZPW = _NPAD // 32
_ZCH = 80
_NZC = _ZPW // _ZCH

_BN = 1024
_GRID = _NPAD // _BN

_mesh = plsc.VectorSubcoreMesh(core_axis_name="c", subcore_axis_name="s")


@functools.partial(
    pl.kernel,
    mesh=_mesh,
    out_type=jax.ShapeDtypeStruct((_NPAD, _H), jnp.float32),
    scratch_types=[
        pltpu.VMEM((_ZCH,), jnp.int32),
        pltpu.VMEM((_ZCH, _H), jnp.float32),
        pltpu.SemaphoreType.DMA,
    ],
)
def _embed(tab_hbm, z_hbm, x_hbm, idx_v, rows_v, sem):
    wid = lax.axis_index("s") * 2 + lax.axis_index("c")
    base0 = wid * _ZPW

    def body(j, carry):
        base = base0 + j * _ZCH
        pltpu.sync_copy(z_hbm.at[pl.ds(base, _ZCH)], idx_v)
        pltpu.async_copy(tab_hbm.at[idx_v], rows_v, sem).wait()
        pltpu.sync_copy(rows_v, x_hbm.at[pl.ds(base, _ZCH), :])
        return carry

    lax.fori_loop(0, _NZC, body, 0)


@functools.partial(
    pl.kernel,
    mesh=_mesh,
    out_type=jax.ShapeDtypeStruct((2 * _NPAD, _H), jnp.float32),
    scratch_types=[
        pltpu.VMEM((_CH,), jnp.int32),
        pltpu.VMEM((_CH,), jnp.int32),
        pltpu.VMEM((_CH,), jnp.int32),
        pltpu.VMEM((_CH,), jnp.int32),
        pltpu.VMEM((_CH, _H), jnp.float32),
        pltpu.VMEM((_CH, _H), jnp.float32),
        pltpu.VMEM((_ZB, _H), jnp.float32),
        pltpu.VMEM_SHARED((_NPAD, _H), jnp.float32),
        pltpu.SemaphoreType.DMA,
        pltpu.SemaphoreType.DMA,
    ],
)
def _scatter(x_hbm, src_hbm, dst_hbm, zeros_hbm, out_hbm,
             sidx0, didx0, sidx1, didx1, rows0, rows1, zbuf, agg,
             sem0, sem1):
    c = lax.axis_index("c")
    s = lax.axis_index("s")
    wid = s * 2 + c
    rbase = s * _RPS
    pltpu.sync_copy(zeros_hbm, zbuf)

    def zbody(j, carry):
        pltpu.sync_copy(zbuf, agg.at[pl.ds(rbase + j * _ZB, _ZB), :])
        return carry

    lax.fori_loop(0, _NZB, zbody, 0)
    plsc.subcore_barrier()

    ebase = wid * _EPW

    def prefetch(chunk, sidx, didx, rows, sem):
        base = ebase + chunk * _CH
        pltpu.sync_copy(src_hbm.at[pl.ds(base, _CH)], sidx)
        pltpu.sync_copy(dst_hbm.at[pl.ds(base, _CH)], didx)
        pltpu.async_copy(x_hbm.at[sidx], rows, sem)

    def drain(sidx, didx, rows, sem):
        pltpu.make_async_copy(x_hbm.at[sidx], rows, sem).wait()
        pltpu.sync_copy(rows, agg.at[didx], add=True)

    prefetch(0, sidx0, didx0, rows0, sem0)

    def body(r, carry):
        prefetch(2 * r + 1, sidx1, didx1, rows1, sem1)
        drain(sidx0, didx0, rows0, sem0)
        prefetch(2 * r + 2, sidx0, didx0, rows0, sem0)
        drain(sidx1, didx1, rows1, sem1)
        return carry

    lax.fori_loop(0, (_NCH - 1) // 2, body, 0)
    drain(sidx0, didx0, rows0, sem0)

    plsc.subcore_barrier()
    obase = c * _NPAD + rbase
    pltpu.sync_copy(agg.at[pl.ds(rbase, _RPS), :],
                    out_hbm.at[pl.ds(obase, _RPS), :])


def _mlp_body(x_ref, agg_ref, w1_ref, b1_ref, w2_ref, b2_ref, sc_ref,
              be_ref, o_ref):
    h = x_ref[...] + agg_ref[0] + agg_ref[1]
    h = jnp.dot(h, w1_ref[...], preferred_element_type=jnp.float32, precision=lax.Precision.HIGHEST) + b1_ref[...]
    h = jnp.maximum(h, 0.0)
    h = jnp.dot(h, w2_ref[...], preferred_element_type=jnp.float32, precision=lax.Precision.HIGHEST) + b2_ref[...]
    h = jnp.maximum(h, 0.0)
    o_ref[...] = h * sc_ref[...] + be_ref[...]


def _mlp(x, agg, w1, b1, w2, b2, scl, be):
    return pl.pallas_call(
        _mlp_body,
        grid=(_GRID,),
        in_specs=[
            pl.BlockSpec((_BN, _H), lambda i: (i, 0)),
            pl.BlockSpec((2, _BN, _H), lambda i: (0, i, 0)),
            pl.BlockSpec((_H, _H), lambda i: (0, 0)),
            pl.BlockSpec((1, _H), lambda i: (0, 0)),
            pl.BlockSpec((_H, _H), lambda i: (0, 0)),
            pl.BlockSpec((1, _H), lambda i: (0, 0)),
            pl.BlockSpec((1, _H), lambda i: (0, 0)),
            pl.BlockSpec((1, _H), lambda i: (0, 0)),
        ],
        out_specs=pl.BlockSpec((_BN, _H), lambda i: (i, 0)),
        out_shape=jax.ShapeDtypeStruct((_NPAD, _H), jnp.float32),
    )(x, agg, w1, b1, w2, b2, scl, be)


def _pool_body(x1_ref, x2_ref, x3_ref, b_ref, w1_ref, b1_ref, w2_ref,
               b2_ref, o_ref, sums, cnt):
    i = pl.program_id(0)

    @pl.when(i == 0)
    def _():
        sums[...] = jnp.zeros((_NG, 3 * _H), jnp.float32)
        cnt[...] = jnp.zeros((_NG, _H), jnp.float32)

    seg = b_ref[0, 0, :]
    oh = (lax.broadcasted_iota(jnp.int32, (_NG, _BN), 0)
          == seg[None, :]).astype(jnp.float32)
    sums[:, 0:_H] += jnp.dot(oh, x1_ref[...], preferred_element_type=jnp.float32, precision=lax.Precision.HIGHEST)
    sums[:, _H:2 * _H] += jnp.dot(oh, x2_ref[...], preferred_element_type=jnp.float32, precision=lax.Precision.HIGHEST)
    sums[:, 2 * _H:3 * _H] += jnp.dot(oh, x3_ref[...], preferred_element_type=jnp.float32, precision=lax.Precision.HIGHEST)
    cnt[...] += jnp.broadcast_to(jnp.sum(oh, axis=1, keepdims=True), (_NG, _H))

    @pl.when(i == _GRID - 1)
    def _():
        c = jnp.maximum(cnt[...], 1.0)
        h = (jnp.dot(sums[:, 0:_H] / c, w1_ref[0:_H, :],
                     preferred_element_type=jnp.float32, precision=lax.Precision.HIGHEST)
             + jnp.dot(sums[:, _H:2 * _H] / c, w1_ref[_H:2 * _H, :],
                       preferred_element_type=jnp.float32, precision=lax.Precision.HIGHEST)
             + jnp.dot(sums[:, 2 * _H:3 * _H] / c, w1_ref[2 * _H:3 * _H, :],
                       preferred_element_type=jnp.float32, precision=lax.Precision.HIGHEST)
             + b1_ref[...])
        h = jnp.maximum(h, 0.0)
        o_ref[...] = jnp.dot(h, w2_ref[...],
                             preferred_element_type=jnp.float32, precision=lax.Precision.HIGHEST) + b2_ref[...]


def _pool(x1, x2, x3, bp, w1, b1, w2p, b2p):
    return pl.pallas_call(
        _pool_body,
        grid=(_GRID,),
        in_specs=[
            pl.BlockSpec((_BN, _H), lambda i: (i, 0)),
            pl.BlockSpec((_BN, _H), lambda i: (i, 0)),
            pl.BlockSpec((_BN, _H), lambda i: (i, 0)),
            pl.BlockSpec((1, 1, _BN), lambda i: (i, 0, 0)),
            pl.BlockSpec((3 * _H, _H), lambda i: (0, 0)),
            pl.BlockSpec((1, _H), lambda i: (0, 0)),
            pl.BlockSpec((_H, _H), lambda i: (0, 0)),
            pl.BlockSpec((1, _H), lambda i: (0, 0)),
        ],
        out_specs=pl.BlockSpec((_NG, _H), lambda i: (0, 0)),
        out_shape=jax.ShapeDtypeStruct((_NG, _H), jnp.float32),
        scratch_shapes=[
            pltpu.VMEM((_NG, 3 * _H), jnp.float32),
            pltpu.VMEM((_NG, _H), jnp.float32),
        ],
    )(x1, x2, x3, bp, w1, b1, w2p, b2p)


def kernel(z, edge_index, batch, z_table, W1_0, b1_0, W2_0, b2_0, g_0, be_0,
           W1_1, b1_1, W2_1, b2_1, g_1, be_1, W1_2, b1_2, W2_2, b2_2, g_2,
           be_2, lin1_W, lin1_b, lin2_W, lin2_b):
    f32 = jnp.float32
    z = z.astype(jnp.int32)
    ei = edge_index.astype(jnp.int32)
    batch = batch.astype(jnp.int32)

    src = jnp.concatenate([ei[0], jnp.zeros((_EP - _E,), jnp.int32)])
    dst = jnp.concatenate([ei[1], jnp.full((_EP - _E,), _NPAD - 1, jnp.int32)])
    zp = jnp.concatenate([z, jnp.zeros((_NPAD - _N,), jnp.int32)])
    zeros_stage = jnp.zeros((_ZB, _H), f32)

    x = _embed(z_table, zp)

    layers = [
        (W1_0, b1_0, W2_0, b2_0, g_0, be_0),
        (W1_1, b1_1, W2_1, b2_1, g_1, be_1),
        (W1_2, b1_2, W2_2, b2_2, g_2, be_2),
    ]
    xs = []
    for (w1, b1, w2, b2, g, be) in layers:
        aggf = _scatter(x, src, dst, zeros_stage)
        agg = aggf.reshape(2, _NPAD, _H)
        scl = (g / jnp.sqrt(1.0 + _BN_EPS)).reshape(1, _H)
        x = _mlp(x, agg, w1, b1.reshape(1, _H), w2, b2.reshape(1, _H),
                 scl, be.reshape(1, _H))
        xs.append(x)

    bp = jnp.concatenate(
        [batch, jnp.full((_NPAD - _N,), _NG, jnp.int32)]).reshape(_GRID, 1, _BN)
    w2p = jnp.pad(lin2_W, ((0, 0), (0, _H - 1)))
    b2p = jnp.pad(lin2_b, (0, _H - 1)).reshape(1, _H)
    out = _pool(xs[0], xs[1], xs[2], bp, lin1_W, lin1_b.reshape(1, _H),
                w2p, b2p)
    return out[:, :1]


# speedup vs baseline: 10.1075x; 2.2940x over previous
import functools

import jax
import jax.numpy as jnp
from jax import lax
from jax.experimental import pallas as pl
from jax.experimental.pallas import tpu as pltpu
from jax.experimental.pallas import tpu_sc as plsc

_N, _E, _H, _NG = 10000, 320000, 128, 64
_BN_EPS = 1e-05

_NPAD = 10240
_CH = 128
_NCH = 80
_EPW = _CH * _NCH
_EP = 32 * _EPW
_RPS = _NPAD // 16
_ZB = 64
_NZB = _RPS // _ZB
_ZPW = _NPAD // 32
_ZCH = 80
_NZC = _ZPW // _ZCH

_BN = 1024
_GRID = _NPAD // _BN

_mesh = plsc.VectorSubcoreMesh(core_axis_name="c", subcore_axis_name="s")


@functools.partial(
    pl.kernel,
    mesh=_mesh,
    out_type=jax.ShapeDtypeStruct((_NPAD, _H), jnp.float32),
    scratch_types=[
        pltpu.VMEM((_ZCH,), jnp.int32),
        pltpu.VMEM((_ZCH, _H), jnp.float32),
        pltpu.SemaphoreType.DMA,
    ],
)
def _embed(tab_hbm, z_hbm, x_hbm, idx_v, rows_v, sem):
    wid = lax.axis_index("s") * 2 + lax.axis_index("c")
    base0 = wid * _ZPW

    def body(j, carry):
        base = base0 + j * _ZCH
        pltpu.sync_copy(z_hbm.at[pl.ds(base, _ZCH)], idx_v)
        pltpu.async_copy(tab_hbm.at[idx_v], rows_v, sem).wait()
        pltpu.sync_copy(rows_v, x_hbm.at[pl.ds(base, _ZCH), :])
        return carry

    lax.fori_loop(0, _NZC, body, 0)


@functools.partial(
    pl.kernel,
    mesh=_mesh,
    out_type=jax.ShapeDtypeStruct((2 * _NPAD, _H), jnp.float32),
    scratch_types=[
        pltpu.VMEM((_CH,), jnp.int32),
        pltpu.VMEM((_CH,), jnp.int32),
        pltpu.VMEM((_CH,), jnp.int32),
        pltpu.VMEM((_CH,), jnp.int32),
        pltpu.VMEM((_CH,), jnp.int32),
        pltpu.VMEM((_CH,), jnp.int32),
        pltpu.VMEM((_CH,), jnp.int32),
        pltpu.VMEM((_CH,), jnp.int32),
        pltpu.VMEM((_CH, _H), jnp.float32),
        pltpu.VMEM((_CH, _H), jnp.float32),
        pltpu.VMEM((_ZB, _H), jnp.float32),
        pltpu.VMEM_SHARED((_NPAD, _H), jnp.float32),
        pltpu.SemaphoreType.DMA,
        pltpu.SemaphoreType.DMA,
        pltpu.SemaphoreType.DMA,
        pltpu.SemaphoreType.DMA,
        pltpu.SemaphoreType.DMA,
        pltpu.SemaphoreType.DMA,
        pltpu.SemaphoreType.DMA,
        pltpu.SemaphoreType.DMA,
    ],
)
def _scatter(x_hbm, src_hbm, dst_hbm, zeros_hbm, out_hbm,
             si0, si1, si2, si3, di0, di1, di2, di3, rows0, rows1, zbuf, agg,
             is0, is1, is2, is3, gs0, gs1, ss0, ss1):
    si = (si0, si1, si2, si3)
    di = (di0, di1, di2, di3)
    rows = (rows0, rows1)
    isem = (is0, is1, is2, is3)
    gsem = (gs0, gs1)
    ssem = (ss0, ss1)

    c = lax.axis_index("c")
    s = lax.axis_index("s")
    wid = s * 2 + c
    rbase = s * _RPS
    pltpu.sync_copy(zeros_hbm, zbuf)

    def zbody(j, carry):
        pltpu.sync_copy(zbuf, agg.at[pl.ds(rbase + j * _ZB, _ZB), :])
        return carry

    lax.fori_loop(0, _NZB, zbody, 0)
    plsc.subcore_barrier()

    ebase = wid * _EPW

    def idx_load(chunk, k):
        base = ebase + chunk * _CH
        pltpu.async_copy(src_hbm.at[pl.ds(base, _CH)], si[k], isem[k])
        pltpu.async_copy(dst_hbm.at[pl.ds(base, _CH)], di[k], isem[k])

    def idx_wait(k):
        pltpu.make_async_copy(src_hbm.at[pl.ds(0, _CH)], si[k], isem[k]).wait()
        pltpu.make_async_copy(dst_hbm.at[pl.ds(0, _CH)], di[k], isem[k]).wait()

    def gather(k, rk):
        pltpu.async_copy(x_hbm.at[si[k]], rows[rk], gsem[rk])

    def gather_wait(k, rk):
        pltpu.make_async_copy(x_hbm.at[si[k]], rows[rk], gsem[rk]).wait()

    def scat(k, rk):
        pltpu.async_copy(rows[rk], agg.at[di[k]], ssem[rk], add=True)

    def scat_wait(k, rk):
        pltpu.make_async_copy(rows[rk], agg.at[di[k]], ssem[rk]).wait()

    def step(cc, j, do_idx=True, do_gather=True):
        scat_wait((j + 3) % 4, (j + 1) % 2)
        if do_idx:
            idx_load(cc + 2, (j + 2) % 4)
        if do_gather:
            idx_wait((j + 1) % 4)
            gather((j + 1) % 4, (j + 1) % 2)
        gather_wait(j % 4, j % 2)
        scat(j % 4, j % 2)

    idx_load(0, 0)
    idx_load(1, 1)
    idx_wait(0)
    gather(0, 0)
    idx_load(2, 2)
    idx_wait(1)
    gather(1, 1)
    gather_wait(0, 0)
    scat(0, 0)

    def body(r, carry):
        cbase = 1 + 4 * r
        for kk in range(4):
            step(cbase + kk, 1 + kk)
        return carry

    lax.fori_loop(0, (_NCH - 4) // 4, body, 0)
    step(_NCH - 3, _NCH - 3)
    step(_NCH - 2, _NCH - 2, do_idx=False)
    step(_NCH - 1, _NCH - 1, do_idx=False, do_gather=False)
    scat_wait((_NCH - 1) % 4, (_NCH - 1) % 2)

    plsc.subcore_barrier()
    obase = c * _NPAD + rbase
    pltpu.sync_copy(agg.at[pl.ds(rbase, _RPS), :],
                    out_hbm.at[pl.ds(obase, _RPS), :])


def _mlp_body(x_ref, agg_ref, w1_ref, b1_ref, w2_ref, b2_ref, sc_ref,
              be_ref, o_ref):
    h = x_ref[...] + agg_ref[0] + agg_ref[1]
    h = jnp.dot(h, w1_ref[...], preferred_element_type=jnp.float32, precision=lax.Precision.HIGHEST) + b1_ref[...]
    h = jnp.maximum(h, 0.0)
    h = jnp.dot(h, w2_ref[...], preferred_element_type=jnp.float32, precision=lax.Precision.HIGHEST) + b2_ref[...]
    h = jnp.maximum(h, 0.0)
    o_ref[...] = h * sc_ref[...] + be_ref[...]


def _mlp(x, agg, w1, b1, w2, b2, scl, be):
    return pl.pallas_call(
        _mlp_body,
        grid=(_GRID,),
        in_specs=[
            pl.BlockSpec((_BN, _H), lambda i: (i, 0)),
            pl.BlockSpec((2, _BN, _H), lambda i: (0, i, 0)),
            pl.BlockSpec((_H, _H), lambda i: (0, 0)),
            pl.BlockSpec((1, _H), lambda i: (0, 0)),
            pl.BlockSpec((_H, _H), lambda i: (0, 0)),
            pl.BlockSpec((1, _H), lambda i: (0, 0)),
            pl.BlockSpec((1, _H), lambda i: (0, 0)),
            pl.BlockSpec((1, _H), lambda i: (0, 0)),
        ],
        out_specs=pl.BlockSpec((_BN, _H), lambda i: (i, 0)),
        out_shape=jax.ShapeDtypeStruct((_NPAD, _H), jnp.float32),
    )(x, agg, w1, b1, w2, b2, scl, be)


def _pool_body(x1_ref, x2_ref, x3_ref, b_ref, w1_ref, b1_ref, w2_ref,
               b2_ref, o_ref, sums, cnt):
    i = pl.program_id(0)

    @pl.when(i == 0)
    def _():
        sums[...] = jnp.zeros((_NG, 3 * _H), jnp.float32)
        cnt[...] = jnp.zeros((_NG, _H), jnp.float32)

    seg = b_ref[0, 0, :]
    oh = (lax.broadcasted_iota(jnp.int32, (_NG, _BN), 0)
          == seg[None, :]).astype(jnp.float32)
    sums[:, 0:_H] += jnp.dot(oh, x1_ref[...], preferred_element_type=jnp.float32, precision=lax.Precision.HIGHEST)
    sums[:, _H:2 * _H] += jnp.dot(oh, x2_ref[...], preferred_element_type=jnp.float32, precision=lax.Precision.HIGHEST)
    sums[:, 2 * _H:3 * _H] += jnp.dot(oh, x3_ref[...], preferred_element_type=jnp.float32, precision=lax.Precision.HIGHEST)
    cnt[...] += jnp.broadcast_to(jnp.sum(oh, axis=1, keepdims=True), (_NG, _H))

    @pl.when(i == _GRID - 1)
    def _():
        c = jnp.maximum(cnt[...], 1.0)
        h = (jnp.dot(sums[:, 0:_H] / c, w1_ref[0:_H, :],
                     preferred_element_type=jnp.float32, precision=lax.Precision.HIGHEST)
             + jnp.dot(sums[:, _H:2 * _H] / c, w1_ref[_H:2 * _H, :],
                       preferred_element_type=jnp.float32, precision=lax.Precision.HIGHEST)
             + jnp.dot(sums[:, 2 * _H:3 * _H] / c, w1_ref[2 * _H:3 * _H, :],
                       preferred_element_type=jnp.float32, precision=lax.Precision.HIGHEST)
             + b1_ref[...])
        h = jnp.maximum(h, 0.0)
        o_ref[...] = jnp.dot(h, w2_ref[...],
                             preferred_element_type=jnp.float32, precision=lax.Precision.HIGHEST) + b2_ref[...]


def _pool(x1, x2, x3, bp, w1, b1, w2p, b2p):
    return pl.pallas_call(
        _pool_body,
        grid=(_GRID,),
        in_specs=[
            pl.BlockSpec((_BN, _H), lambda i: (i, 0)),
            pl.BlockSpec((_BN, _H), lambda i: (i, 0)),
            pl.BlockSpec((_BN, _H), lambda i: (i, 0)),
            pl.BlockSpec((1, 1, _BN), lambda i: (i, 0, 0)),
            pl.BlockSpec((3 * _H, _H), lambda i: (0, 0)),
            pl.BlockSpec((1, _H), lambda i: (0, 0)),
            pl.BlockSpec((_H, _H), lambda i: (0, 0)),
            pl.BlockSpec((1, _H), lambda i: (0, 0)),
        ],
        out_specs=pl.BlockSpec((_NG, _H), lambda i: (0, 0)),
        out_shape=jax.ShapeDtypeStruct((_NG, _H), jnp.float32),
        scratch_shapes=[
            pltpu.VMEM((_NG, 3 * _H), jnp.float32),
            pltpu.VMEM((_NG, _H), jnp.float32),
        ],
    )(x1, x2, x3, bp, w1, b1, w2p, b2p)


def kernel(z, edge_index, batch, z_table, W1_0, b1_0, W2_0, b2_0, g_0, be_0,
           W1_1, b1_1, W2_1, b2_1, g_1, be_1, W1_2, b1_2, W2_2, b2_2, g_2,
           be_2, lin1_W, lin1_b, lin2_W, lin2_b):
    f32 = jnp.float32
    z = z.astype(jnp.int32)
    ei = edge_index.astype(jnp.int32)
    batch = batch.astype(jnp.int32)

    npd = _EP - _E
    src = jnp.concatenate([ei[0], (jnp.arange(npd, dtype=jnp.int32) * 7919) % _N])
    dst = jnp.concatenate(
        [ei[1], _N + (jnp.arange(npd, dtype=jnp.int32) % (_NPAD - _N))])
    zp = jnp.concatenate([z, jnp.zeros((_NPAD - _N,), jnp.int32)])
    zeros_stage = jnp.zeros((_ZB, _H), f32)

    x = _embed(z_table, zp)

    layers = [
        (W1_0, b1_0, W2_0, b2_0, g_0, be_0),
        (W1_1, b1_1, W2_1, b2_1, g_1, be_1),
        (W1_2, b1_2, W2_2, b2_2, g_2, be_2),
    ]
    xs = []
    for (w1, b1, w2, b2, g, be) in layers:
        aggf = _scatter(x, src, dst, zeros_stage)
        agg = aggf.reshape(2, _NPAD, _H)
        scl = (g / jnp.sqrt(1.0 + _BN_EPS)).reshape(1, _H)
        x = _mlp(x, agg, w1, b1.reshape(1, _H), w2, b2.reshape(1, _H),
                 scl, be.reshape(1, _H))
        xs.append(x)

    bp = jnp.concatenate(
        [batch, jnp.full((_NPAD - _N,), _NG, jnp.int32)]).reshape(_GRID, 1, _BN)
    w2p = jnp.pad(lin2_W, ((0, 0), (0, _H - 1)))
    b2p = jnp.pad(lin2_b, (0, _H - 1)).reshape(1, _H)
    out = _pool(xs[0], xs[1], xs[2], bp, lin1_W, lin1_b.reshape(1, _H),
                w2p, b2p)
    return out[:, :1]


# speedup vs baseline: 10.2581x; 1.0149x over previous
import functools

import jax
import jax.numpy as jnp
from jax import lax
from jax.experimental import pallas as pl
from jax.experimental.pallas import tpu as pltpu
from jax.experimental.pallas import tpu_sc as plsc

_N, _E, _H, _NG = 10000, 320000, 128, 64
_BN_EPS = 1e-05

_NPAD = 10240
_CH = 128
_NCH = 80
_EPW = _CH * _NCH
_EP = 32 * _EPW
_RPS = _NPAD // 16
_ZB = 64
_NZB = _RPS // _ZB
_ZPW = _NPAD // 32
_ZCH = 80
_NZC = _ZPW // _ZCH

_BN = 1024
_GRID = _NPAD // _BN

_mesh = plsc.VectorSubcoreMesh(core_axis_name="c", subcore_axis_name="s")


@functools.partial(
    pl.kernel,
    mesh=_mesh,
    out_type=jax.ShapeDtypeStruct((_NPAD, _H), jnp.float32),
    scratch_types=[
        pltpu.VMEM((_ZCH,), jnp.int32),
        pltpu.VMEM((_ZCH,), jnp.int32),
        pltpu.VMEM((_ZCH, _H), jnp.float32),
        pltpu.VMEM((_ZCH, _H), jnp.float32),
        pltpu.SemaphoreType.DMA,
        pltpu.SemaphoreType.DMA,
        pltpu.SemaphoreType.DMA,
        pltpu.SemaphoreType.DMA,
    ],
)
def _embed(tab_hbm, z_hbm, x_hbm, i0, i1, r0, r1, g0, g1, w0, w1):
    wid = lax.axis_index("s") * 2 + lax.axis_index("c")
    base0 = wid * _ZPW
    idx = (i0, i1)
    rows = (r0, r1)
    gsem = (g0, g1)
    wsem = (w0, w1)

    def ld(j, b):
        pltpu.sync_copy(z_hbm.at[pl.ds(base0 + j * _ZCH, _ZCH)], idx[b])
        pltpu.async_copy(tab_hbm.at[idx[b]], rows[b], gsem[b])

    def wb(j, b):
        pltpu.make_async_copy(tab_hbm.at[idx[b]], rows[b], gsem[b]).wait()
        pltpu.async_copy(rows[b],
                         x_hbm.at[pl.ds(base0 + j * _ZCH, _ZCH), :], wsem[b])

    def wb_wait(j, b):
        pltpu.make_async_copy(
            rows[b], x_hbm.at[pl.ds(base0 + j * _ZCH, _ZCH), :],
            wsem[b]).wait()

    ld(0, 0)
    ld(1, 1)
    wb(0, 0)
    wb(1, 1)
    wb_wait(0, 0)
    ld(2, 0)
    wb_wait(1, 1)
    ld(3, 1)
    wb(2, 0)
    wb(3, 1)
    wb_wait(2, 0)
    wb_wait(3, 1)


@functools.partial(
    pl.kernel,
    mesh=_mesh,
    out_type=jax.ShapeDtypeStruct((2 * _NPAD, _H), jnp.float32),
    scratch_types=[
        pltpu.VMEM((_CH,), jnp.int32),
        pltpu.VMEM((_CH,), jnp.int32),
        pltpu.VMEM((_CH,), jnp.int32),
        pltpu.VMEM((_CH,), jnp.int32),
        pltpu.VMEM((_CH,), jnp.int32),
        pltpu.VMEM((_CH,), jnp.int32),
        pltpu.VMEM((_CH,), jnp.int32),
        pltpu.VMEM((_CH,), jnp.int32),
        pltpu.VMEM((_CH, _H), jnp.float32),
        pltpu.VMEM((_CH, _H), jnp.float32),
        pltpu.VMEM((_ZB, _H), jnp.float32),
        pltpu.VMEM_SHARED((_NPAD, _H), jnp.float32),
        pltpu.SemaphoreType.DMA,
        pltpu.SemaphoreType.DMA,
        pltpu.SemaphoreType.DMA,
        pltpu.SemaphoreType.DMA,
        pltpu.SemaphoreType.DMA,
        pltpu.SemaphoreType.DMA,
        pltpu.SemaphoreType.DMA,
        pltpu.SemaphoreType.DMA,
    ],
)
def _scatter(x_hbm, src_hbm, dst_hbm, zeros_hbm, out_hbm,
             si0, si1, si2, si3, di0, di1, di2, di3, rows0, rows1, zbuf, agg,
             is0, is1, is2, is3, gs0, gs1, ss0, ss1):
    si = (si0, si1, si2, si3)
    di = (di0, di1, di2, di3)
    rows = (rows0, rows1)
    isem = (is0, is1, is2, is3)
    gsem = (gs0, gs1)
    ssem = (ss0, ss1)

    c = lax.axis_index("c")
    s = lax.axis_index("s")
    wid = s * 2 + c
    rbase = s * _RPS
    ebase = wid * _EPW
    pltpu.async_copy(zeros_hbm, zbuf, ss0)

    def idx_load(chunk, k):
        base = ebase + chunk * _CH
        pltpu.async_copy(src_hbm.at[pl.ds(base, _CH)], si[k], isem[k])
        pltpu.async_copy(dst_hbm.at[pl.ds(base, _CH)], di[k], isem[k])

    def idx_wait(k):
        pltpu.make_async_copy(src_hbm.at[pl.ds(0, _CH)], si[k], isem[k]).wait()
        pltpu.make_async_copy(dst_hbm.at[pl.ds(0, _CH)], di[k], isem[k]).wait()

    def gather(k, rk):
        pltpu.async_copy(x_hbm.at[si[k]], rows[rk], gsem[rk])

    def gather_wait(k, rk):
        pltpu.make_async_copy(x_hbm.at[si[k]], rows[rk], gsem[rk]).wait()

    def scat(k, rk):
        pltpu.async_copy(rows[rk], agg.at[di[k]], ssem[rk], add=True)

    def scat_wait(k, rk):
        pltpu.make_async_copy(rows[rk], agg.at[di[k]], ssem[rk]).wait()

    def step(cc, j, do_idx=True, do_gather=True):
        scat_wait((j + 3) % 4, (j + 1) % 2)
        if do_idx:
            idx_load(cc + 2, (j + 2) % 4)
        if do_gather:
            idx_wait((j + 1) % 4)
            gather((j + 1) % 4, (j + 1) % 2)
        gather_wait(j % 4, j % 2)
        scat(j % 4, j % 2)

    idx_load(0, 0)
    idx_load(1, 1)
    idx_wait(0)
    gather(0, 0)
    idx_load(2, 2)
    idx_wait(1)
    gather(1, 1)

    pltpu.make_async_copy(zeros_hbm, zbuf, ss0).wait()

    def zbody(j, carry):
        pltpu.sync_copy(zbuf, agg.at[pl.ds(rbase + j * _ZB, _ZB), :])
        return carry

    lax.fori_loop(0, _NZB, zbody, 0)
    plsc.subcore_barrier()

    gather_wait(0, 0)
    scat(0, 0)

    def body(r, carry):
        cbase = 1 + 4 * r
        for kk in range(4):
            step(cbase + kk, 1 + kk)
        return carry

    lax.fori_loop(0, (_NCH - 4) // 4, body, 0)
    step(_NCH - 3, _NCH - 3)
    step(_NCH - 2, _NCH - 2, do_idx=False)
    step(_NCH - 1, _NCH - 1, do_idx=False, do_gather=False)
    scat_wait((_NCH - 1) % 4, (_NCH - 1) % 2)

    plsc.subcore_barrier()
    obase = c * _NPAD + rbase
    pltpu.sync_copy(agg.at[pl.ds(rbase, _RPS), :],
                    out_hbm.at[pl.ds(obase, _RPS), :])


def _mlp_body(x_ref, agg_ref, w1_ref, b1_ref, w2_ref, b2_ref, sc_ref,
              be_ref, o_ref):
    h = x_ref[...] + agg_ref[0] + agg_ref[1]
    h = jnp.dot(h, w1_ref[...], preferred_element_type=jnp.float32, precision=lax.Precision.HIGHEST) + b1_ref[...]
    h = jnp.maximum(h, 0.0)
    h = jnp.dot(h, w2_ref[...], preferred_element_type=jnp.float32, precision=lax.Precision.HIGHEST) + b2_ref[...]
    h = jnp.maximum(h, 0.0)
    o_ref[...] = h * sc_ref[...] + be_ref[...]


def _mlp(x, agg, w1, b1, w2, b2, scl, be):
    return pl.pallas_call(
        _mlp_body,
        grid=(_GRID,),
        in_specs=[
            pl.BlockSpec((_BN, _H), lambda i: (i, 0)),
            pl.BlockSpec((2, _BN, _H), lambda i: (0, i, 0)),
            pl.BlockSpec((_H, _H), lambda i: (0, 0)),
            pl.BlockSpec((1, _H), lambda i: (0, 0)),
            pl.BlockSpec((_H, _H), lambda i: (0, 0)),
            pl.BlockSpec((1, _H), lambda i: (0, 0)),
            pl.BlockSpec((1, _H), lambda i: (0, 0)),
            pl.BlockSpec((1, _H), lambda i: (0, 0)),
        ],
        out_specs=pl.BlockSpec((_BN, _H), lambda i: (i, 0)),
        out_shape=jax.ShapeDtypeStruct((_NPAD, _H), jnp.float32),
    )(x, agg, w1, b1, w2, b2, scl, be)


def _pool_body(x1_ref, x2_ref, x3_ref, b_ref, w1_ref, b1_ref, w2_ref,
               b2_ref, o_ref, sums, cnt):
    i = pl.program_id(0)

    @pl.when(i == 0)
    def _():
        sums[...] = jnp.zeros((_NG, 3 * _H), jnp.float32)
        cnt[...] = jnp.zeros((_NG, _H), jnp.float32)

    seg = b_ref[0, 0, :]
    oh = (lax.broadcasted_iota(jnp.int32, (_NG, _BN), 0)
          == seg[None, :]).astype(jnp.float32)
    sums[:, 0:_H] += jnp.dot(oh, x1_ref[...], preferred_element_type=jnp.float32, precision=lax.Precision.HIGHEST)
    sums[:, _H:2 * _H] += jnp.dot(oh, x2_ref[...], preferred_element_type=jnp.float32, precision=lax.Precision.HIGHEST)
    sums[:, 2 * _H:3 * _H] += jnp.dot(oh, x3_ref[...], preferred_element_type=jnp.float32, precision=lax.Precision.HIGHEST)
    cnt[...] += jnp.broadcast_to(jnp.sum(oh, axis=1, keepdims=True), (_NG, _H))

    @pl.when(i == _GRID - 1)
    def _():
        c = jnp.maximum(cnt[...], 1.0)
        h = (jnp.dot(sums[:, 0:_H] / c, w1_ref[0:_H, :],
                     preferred_element_type=jnp.float32, precision=lax.Precision.HIGHEST)
             + jnp.dot(sums[:, _H:2 * _H] / c, w1_ref[_H:2 * _H, :],
                       preferred_element_type=jnp.float32, precision=lax.Precision.HIGHEST)
             + jnp.dot(sums[:, 2 * _H:3 * _H] / c, w1_ref[2 * _H:3 * _H, :],
                       preferred_element_type=jnp.float32, precision=lax.Precision.HIGHEST)
             + b1_ref[...])
        h = jnp.maximum(h, 0.0)
        o_ref[...] = jnp.dot(h, w2_ref[...],
                             preferred_element_type=jnp.float32, precision=lax.Precision.HIGHEST) + b2_ref[...]


def _pool(x1, x2, x3, bp, w1, b1, w2p, b2p):
    return pl.pallas_call(
        _pool_body,
        grid=(_GRID,),
        in_specs=[
            pl.BlockSpec((_BN, _H), lambda i: (i, 0)),
            pl.BlockSpec((_BN, _H), lambda i: (i, 0)),
            pl.BlockSpec((_BN, _H), lambda i: (i, 0)),
            pl.BlockSpec((1, 1, _BN), lambda i: (i, 0, 0)),
            pl.BlockSpec((3 * _H, _H), lambda i: (0, 0)),
            pl.BlockSpec((1, _H), lambda i: (0, 0)),
            pl.BlockSpec((_H, _H), lambda i: (0, 0)),
            pl.BlockSpec((1, _H), lambda i: (0, 0)),
        ],
        out_specs=pl.BlockSpec((_NG, _H), lambda i: (0, 0)),
        out_shape=jax.ShapeDtypeStruct((_NG, _H), jnp.float32),
        scratch_shapes=[
            pltpu.VMEM((_NG, 3 * _H), jnp.float32),
            pltpu.VMEM((_NG, _H), jnp.float32),
        ],
    )(x1, x2, x3, bp, w1, b1, w2p, b2p)


def kernel(z, edge_index, batch, z_table, W1_0, b1_0, W2_0, b2_0, g_0, be_0,
           W1_1, b1_1, W2_1, b2_1, g_1, be_1, W1_2, b1_2, W2_2, b2_2, g_2,
           be_2, lin1_W, lin1_b, lin2_W, lin2_b):
    f32 = jnp.float32
    z = z.astype(jnp.int32)
    ei = edge_index.astype(jnp.int32)
    batch = batch.astype(jnp.int32)

    npd = _EP - _E
    src = jnp.concatenate([ei[0], (jnp.arange(npd, dtype=jnp.int32) * 7919) % _N])
    dst = jnp.concatenate(
        [ei[1], _N + (jnp.arange(npd, dtype=jnp.int32) % (_NPAD - _N))])
    zp = jnp.concatenate([z, jnp.zeros((_NPAD - _N,), jnp.int32)])
    zeros_stage = jnp.zeros((_ZB, _H), f32)

    x = _embed(z_table, zp)

    layers = [
        (W1_0, b1_0, W2_0, b2_0, g_0, be_0),
        (W1_1, b1_1, W2_1, b2_1, g_1, be_1),
        (W1_2, b1_2, W2_2, b2_2, g_2, be_2),
    ]
    xs = []
    for (w1, b1, w2, b2, g, be) in layers:
        aggf = _scatter(x, src, dst, zeros_stage)
        agg = aggf.reshape(2, _NPAD, _H)
        scl = (g / jnp.sqrt(1.0 + _BN_EPS)).reshape(1, _H)
        x = _mlp(x, agg, w1, b1.reshape(1, _H), w2, b2.reshape(1, _H),
                 scl, be.reshape(1, _H))
        xs.append(x)

    bp = jnp.concatenate(
        [batch, jnp.full((_NPAD - _N,), _NG, jnp.int32)]).reshape(_GRID, 1, _BN)
    w2p = jnp.pad(lin2_W, ((0, 0), (0, _H - 1)))
    b2p = jnp.pad(lin2_b, (0, _H - 1)).reshape(1, _H)
    out = _pool(xs[0], xs[1], xs[2], bp, lin1_W, lin1_b.reshape(1, _H),
                w2p, b2p)
    return out[:, :1]


# speedup vs baseline: 10.9765x; 1.0700x over previous
import functools

import jax
import jax.numpy as jnp
from jax import lax
from jax.experimental import pallas as pl
from jax.experimental.pallas import tpu as pltpu
from jax.experimental.pallas import tpu_sc as plsc

_N, _E, _H, _NG = 10000, 320000, 128, 64
_BN_EPS = 1e-05

_NPAD = 10240
_CH = 128
_NCH = 80
_EPW = _CH * _NCH
_EP = 32 * _EPW
_RPS = _NPAD // 16
_ZB = 64
_NZB = _RPS // _ZB
_ZPW = _NPAD // 32
_ZCH = 80
_NZC = _ZPW // _ZCH

_BN = 1024
_GRID = _NPAD // _BN

_mesh = plsc.VectorSubcoreMesh(core_axis_name="c", subcore_axis_name="s")


@functools.partial(
    pl.kernel,
    mesh=_mesh,
    out_type=jax.ShapeDtypeStruct((_NPAD, _H), jnp.float32),
    scratch_types=[
        pltpu.VMEM((_ZCH,), jnp.int32),
        pltpu.VMEM((_ZCH,), jnp.int32),
        pltpu.VMEM((_ZCH, _H), jnp.float32),
        pltpu.VMEM((_ZCH, _H), jnp.float32),
        pltpu.SemaphoreType.DMA,
        pltpu.SemaphoreType.DMA,
        pltpu.SemaphoreType.DMA,
        pltpu.SemaphoreType.DMA,
    ],
)
def _embed(tab_hbm, z_hbm, x_hbm, i0, i1, r0, r1, g0, g1, w0, w1):
    wid = lax.axis_index("s") * 2 + lax.axis_index("c")
    base0 = wid * _ZPW
    idx = (i0, i1)
    rows = (r0, r1)
    gsem = (g0, g1)
    wsem = (w0, w1)

    def ld(j, b):
        pltpu.sync_copy(z_hbm.at[pl.ds(base0 + j * _ZCH, _ZCH)], idx[b])
        pltpu.async_copy(tab_hbm.at[idx[b]], rows[b], gsem[b])

    def wb(j, b):
        pltpu.make_async_copy(tab_hbm.at[idx[b]], rows[b], gsem[b]).wait()
        pltpu.async_copy(rows[b],
                         x_hbm.at[pl.ds(base0 + j * _ZCH, _ZCH), :], wsem[b])

    def wb_wait(j, b):
        pltpu.make_async_copy(
            rows[b], x_hbm.at[pl.ds(base0 + j * _ZCH, _ZCH), :],
            wsem[b]).wait()

    ld(0, 0)
    ld(1, 1)
    wb(0, 0)
    wb(1, 1)
    wb_wait(0, 0)
    ld(2, 0)
    wb_wait(1, 1)
    ld(3, 1)
    wb(2, 0)
    wb(3, 1)
    wb_wait(2, 0)
    wb_wait(3, 1)


@functools.partial(
    pl.kernel,
    mesh=_mesh,
    out_type=jax.ShapeDtypeStruct((2 * _NPAD, _H), jnp.float32),
    scratch_types=[
        pltpu.VMEM((_CH,), jnp.int32),
        pltpu.VMEM((_CH,), jnp.int32),
        pltpu.VMEM((_CH,), jnp.int32),
        pltpu.VMEM((_CH,), jnp.int32),
        pltpu.VMEM((_CH,), jnp.int32),
        pltpu.VMEM((_CH,), jnp.int32),
        pltpu.VMEM((_CH,), jnp.int32),
        pltpu.VMEM((_CH,), jnp.int32),
        pltpu.VMEM((_CH, _H), jnp.float32),
        pltpu.VMEM((_CH, _H), jnp.float32),
        pltpu.VMEM((_ZB, _H), jnp.float32),
        pltpu.VMEM_SHARED((_NPAD, _H), jnp.float32),
        pltpu.SemaphoreType.DMA,
        pltpu.SemaphoreType.DMA,
        pltpu.SemaphoreType.DMA,
        pltpu.SemaphoreType.DMA,
        pltpu.SemaphoreType.DMA,
        pltpu.SemaphoreType.DMA,
        pltpu.SemaphoreType.DMA,
        pltpu.SemaphoreType.DMA,
    ],
)
def _scatter(x_hbm, src_hbm, dst_hbm, zeros_hbm, out_hbm,
             si0, si1, si2, si3, di0, di1, di2, di3, rows0, rows1, zbuf, agg,
             is0, is1, is2, is3, gs0, gs1, ss0, ss1):
    si = (si0, si1, si2, si3)
    di = (di0, di1, di2, di3)
    rows = (rows0, rows1)
    isem = (is0, is1, is2, is3)
    gsem = (gs0, gs1)
    ssem = (ss0, ss1)

    c = lax.axis_index("c")
    s = lax.axis_index("s")
    wid = s * 2 + c
    rbase = s * _RPS
    ebase = wid * _EPW
    pltpu.async_copy(zeros_hbm, zbuf, ss0)

    def idx_load(chunk, k):
        base = ebase + chunk * _CH
        pltpu.async_copy(src_hbm.at[pl.ds(base, _CH)], si[k], isem[k])
        pltpu.async_copy(dst_hbm.at[pl.ds(base, _CH)], di[k], isem[k])

    def idx_wait(k):
        pltpu.make_async_copy(src_hbm.at[pl.ds(0, _CH)], si[k], isem[k]).wait()
        pltpu.make_async_copy(dst_hbm.at[pl.ds(0, _CH)], di[k], isem[k]).wait()

    def gather(k, rk):
        pltpu.async_copy(x_hbm.at[si[k]], rows[rk], gsem[rk])

    def gather_wait(k, rk):
        pltpu.make_async_copy(x_hbm.at[si[k]], rows[rk], gsem[rk]).wait()

    def scat(k, rk):
        pltpu.async_copy(rows[rk], agg.at[di[k]], ssem[rk], add=True)

    def scat_wait(k, rk):
        pltpu.make_async_copy(rows[rk], agg.at[di[k]], ssem[rk]).wait()

    def step(cc, j, do_idx=True, do_gather=True):
        scat_wait((j + 3) % 4, (j + 1) % 2)
        if do_idx:
            idx_load(cc + 2, (j + 2) % 4)
        if do_gather:
            idx_wait((j + 1) % 4)
            gather((j + 1) % 4, (j + 1) % 2)
        gather_wait(j % 4, j % 2)
        scat(j % 4, j % 2)

    idx_load(0, 0)
    idx_load(1, 1)
    idx_wait(0)
    gather(0, 0)
    idx_load(2, 2)
    idx_wait(1)
    gather(1, 1)

    pltpu.make_async_copy(zeros_hbm, zbuf, ss0).wait()

    def zbody(j, carry):
        pltpu.sync_copy(zbuf, agg.at[pl.ds(rbase + j * _ZB, _ZB), :])
        return carry

    lax.fori_loop(0, _NZB, zbody, 0)
    plsc.subcore_barrier()

    gather_wait(0, 0)
    scat(0, 0)

    def body(r, carry):
        cbase = 1 + 4 * r
        for kk in range(4):
            step(cbase + kk, 1 + kk)
        return carry

    lax.fori_loop(0, (_NCH - 4) // 4, body, 0)
    step(_NCH - 3, _NCH - 3)
    step(_NCH - 2, _NCH - 2, do_idx=False)
    step(_NCH - 1, _NCH - 1, do_idx=False, do_gather=False)
    scat_wait((_NCH - 1) % 4, (_NCH - 1) % 2)

    plsc.subcore_barrier()
    obase = c * _NPAD + rbase
    pltpu.sync_copy(agg.at[pl.ds(rbase, _RPS), :],
                    out_hbm.at[pl.ds(obase, _RPS), :])


def _mlp_body(x_ref, agg_ref, w1_ref, b1_ref, w2_ref, b2_ref, sc_ref,
              be_ref, o_ref):
    h = x_ref[...] + agg_ref[0] + agg_ref[1]
    h = jnp.dot(h, w1_ref[...], preferred_element_type=jnp.float32) + b1_ref[...]
    h = jnp.maximum(h, 0.0)
    h = jnp.dot(h, w2_ref[...], preferred_element_type=jnp.float32) + b2_ref[...]
    h = jnp.maximum(h, 0.0)
    o_ref[...] = h * sc_ref[...] + be_ref[...]


def _mlp(x, agg, w1, b1, w2, b2, scl, be):
    return pl.pallas_call(
        _mlp_body,
        grid=(_GRID,),
        in_specs=[
            pl.BlockSpec((_BN, _H), lambda i: (i, 0)),
            pl.BlockSpec((2, _BN, _H), lambda i: (0, i, 0)),
            pl.BlockSpec((_H, _H), lambda i: (0, 0)),
            pl.BlockSpec((1, _H), lambda i: (0, 0)),
            pl.BlockSpec((_H, _H), lambda i: (0, 0)),
            pl.BlockSpec((1, _H), lambda i: (0, 0)),
            pl.BlockSpec((1, _H), lambda i: (0, 0)),
            pl.BlockSpec((1, _H), lambda i: (0, 0)),
        ],
        out_specs=pl.BlockSpec((_BN, _H), lambda i: (i, 0)),
        out_shape=jax.ShapeDtypeStruct((_NPAD, _H), jnp.float32),
    )(x, agg, w1, b1, w2, b2, scl, be)


def _pool_body(x1_ref, x2_ref, x3_ref, b_ref, w1_ref, b1_ref, w2_ref,
               b2_ref, o_ref, sums, cnt):
    i = pl.program_id(0)

    @pl.when(i == 0)
    def _():
        sums[...] = jnp.zeros((_NG, 3 * _H), jnp.float32)
        cnt[...] = jnp.zeros((_NG, _H), jnp.float32)

    seg = b_ref[0, 0, :]
    oh = (lax.broadcasted_iota(jnp.int32, (_NG, _BN), 0)
          == seg[None, :]).astype(jnp.float32)
    sums[:, 0:_H] += jnp.dot(oh, x1_ref[...], preferred_element_type=jnp.float32)
    sums[:, _H:2 * _H] += jnp.dot(oh, x2_ref[...], preferred_element_type=jnp.float32)
    sums[:, 2 * _H:3 * _H] += jnp.dot(oh, x3_ref[...], preferred_element_type=jnp.float32)
    cnt[...] += jnp.broadcast_to(jnp.sum(oh, axis=1, keepdims=True), (_NG, _H))

    @pl.when(i == _GRID - 1)
    def _():
        c = jnp.maximum(cnt[...], 1.0)
        h = (jnp.dot(sums[:, 0:_H] / c, w1_ref[0:_H, :],
                     preferred_element_type=jnp.float32)
             + jnp.dot(sums[:, _H:2 * _H] / c, w1_ref[_H:2 * _H, :],
                       preferred_element_type=jnp.float32)
             + jnp.dot(sums[:, 2 * _H:3 * _H] / c, w1_ref[2 * _H:3 * _H, :],
                       preferred_element_type=jnp.float32)
             + b1_ref[...])
        h = jnp.maximum(h, 0.0)
        o_ref[...] = jnp.dot(h, w2_ref[...],
                             preferred_element_type=jnp.float32) + b2_ref[...]


def _pool(x1, x2, x3, bp, w1, b1, w2p, b2p):
    return pl.pallas_call(
        _pool_body,
        grid=(_GRID,),
        in_specs=[
            pl.BlockSpec((_BN, _H), lambda i: (i, 0)),
            pl.BlockSpec((_BN, _H), lambda i: (i, 0)),
            pl.BlockSpec((_BN, _H), lambda i: (i, 0)),
            pl.BlockSpec((1, 1, _BN), lambda i: (i, 0, 0)),
            pl.BlockSpec((3 * _H, _H), lambda i: (0, 0)),
            pl.BlockSpec((1, _H), lambda i: (0, 0)),
            pl.BlockSpec((_H, _H), lambda i: (0, 0)),
            pl.BlockSpec((1, _H), lambda i: (0, 0)),
        ],
        out_specs=pl.BlockSpec((_NG, _H), lambda i: (0, 0)),
        out_shape=jax.ShapeDtypeStruct((_NG, _H), jnp.float32),
        scratch_shapes=[
            pltpu.VMEM((_NG, 3 * _H), jnp.float32),
            pltpu.VMEM((_NG, _H), jnp.float32),
        ],
    )(x1, x2, x3, bp, w1, b1, w2p, b2p)


def kernel(z, edge_index, batch, z_table, W1_0, b1_0, W2_0, b2_0, g_0, be_0,
           W1_1, b1_1, W2_1, b2_1, g_1, be_1, W1_2, b1_2, W2_2, b2_2, g_2,
           be_2, lin1_W, lin1_b, lin2_W, lin2_b):
    f32 = jnp.float32
    z = z.astype(jnp.int32)
    ei = edge_index.astype(jnp.int32)
    batch = batch.astype(jnp.int32)

    npd = _EP - _E
    src = jnp.concatenate([ei[0], (jnp.arange(npd, dtype=jnp.int32) * 7919) % _N])
    dst = jnp.concatenate(
        [ei[1], _N + (jnp.arange(npd, dtype=jnp.int32) % (_NPAD - _N))])
    zp = jnp.concatenate([z, jnp.zeros((_NPAD - _N,), jnp.int32)])
    zeros_stage = jnp.zeros((_ZB, _H), f32)

    x = _embed(z_table, zp)

    layers = [
        (W1_0, b1_0, W2_0, b2_0, g_0, be_0),
        (W1_1, b1_1, W2_1, b2_1, g_1, be_1),
        (W1_2, b1_2, W2_2, b2_2, g_2, be_2),
    ]
    xs = []
    for (w1, b1, w2, b2, g, be) in layers:
        aggf = _scatter(x, src, dst, zeros_stage)
        agg = aggf.reshape(2, _NPAD, _H)
        scl = (g / jnp.sqrt(1.0 + _BN_EPS)).reshape(1, _H)
        x = _mlp(x, agg, w1, b1.reshape(1, _H), w2, b2.reshape(1, _H),
                 scl, be.reshape(1, _H))
        xs.append(x)

    bp = jnp.concatenate(
        [batch, jnp.full((_NPAD - _N,), _NG, jnp.int32)]).reshape(_GRID, 1, _BN)
    w2p = jnp.pad(lin2_W, ((0, 0), (0, _H - 1)))
    b2p = jnp.pad(lin2_b, (0, _H - 1)).reshape(1, _H)
    out = _pool(xs[0], xs[1], xs[2], bp, lin1_W, lin1_b.reshape(1, _H),
                w2p, b2p)
    return out[:, :1]
